# baseline (speedup 1.0000x reference)
import functools

import jax
import jax.numpy as jnp
from jax import lax
from jax.experimental import pallas as pl
from jax.experimental.pallas import tpu as pltpu
from jax.experimental.pallas import tpu_sc as plsc

E = 320000
D = 128
NCAT = 14
T_ROWS = NCAT * NCAT * NCAT
T_PAD = 2816

NC = 2
NS = 16
NW = NC * NS
R_PER_W = E // NW
GROUP = 400
N_GROUPS = R_PER_W // GROUP
DMA_B = 80
N_DMA = GROUP // DMA_B

IDX_BE = 2000
IDX_GRID = E // IDX_BE


def _build_table(e0, e1, e2):
    def body(e0_ref, e1_ref, e2_ref, t_ref):
        r = lax.broadcasted_iota(jnp.int32, (T_PAD, 1), 0)
        i0 = r // (NCAT * NCAT)
        i1 = (r // NCAT) % NCAT
        i2 = r % NCAT

        def pick(ref, idx):
            acc = jnp.broadcast_to(ref[NCAT - 1:NCAT, :], (T_PAD, D))
            for k in range(NCAT - 2, -1, -1):
                acc = jnp.where(idx == k, ref[k:k + 1, :], acc)
            return acc

        t_ref[...] = (pick(e0_ref, i0) + pick(e1_ref, i1)) + pick(e2_ref, i2)

    return pl.pallas_call(
        body,
        out_shape=jax.ShapeDtypeStruct((T_PAD, D), jnp.float32),
    )(e0, e1, e2)


def _fuse_index(ea3):
    def body(ea_ref, c_ref):
        x = ea_ref[...]
        c_ref[...] = (
            x[..., 0] * (NCAT * NCAT) + x[..., 1] * NCAT + x[..., 2]
        )

    return pl.pallas_call(
        body,
        grid=(IDX_GRID // 8,),
        in_specs=[pl.BlockSpec((8, IDX_BE, 3), lambda i: (i, 0, 0))],
        out_specs=pl.BlockSpec((8, IDX_BE), lambda i: (i, 0)),
        out_shape=jax.ShapeDtypeStruct((IDX_GRID, IDX_BE), jnp.int32),
    )(ea3)


_mesh = plsc.VectorSubcoreMesh(core_axis_name="c", subcore_axis_name="s")


@functools.partial(
    pl.kernel,
    mesh=_mesh,
    out_type=jax.ShapeDtypeStruct((E, D), jnp.float32),
    scratch_types=[
        pltpu.VMEM((GROUP,), jnp.int32),
        pltpu.VMEM((GROUP,), jnp.int32),
        pltpu.VMEM((GROUP, D), jnp.float32),
        pltpu.VMEM((GROUP, D), jnp.float32),
        pltpu.SemaphoreType.DMA,
        pltpu.SemaphoreType.DMA,
        pltpu.SemaphoreType.DMA,
        pltpu.SemaphoreType.DMA,
        pltpu.SemaphoreType.DMA,
        pltpu.SemaphoreType.DMA,
        pltpu.VMEM_SHARED((T_PAD, D), jnp.float32),
    ],
)
def _sc_lookup(c_hbm, t_hbm, out_hbm,
               c0, c1, r0, r1, l0, l1, g0, g1, s0, s1, t_sh):
    sid = lax.axis_index("s")
    wid = sid * NC + lax.axis_index("c")
    base = wid * R_PER_W

    stripe = T_PAD // NS
    pltpu.sync_copy(t_hbm.at[pl.ds(sid * stripe, stripe)],
                    t_sh.at[pl.ds(sid * stripe, stripe)])
    plsc.subcore_barrier()
    cbufs = (c0, c1)
    rbufs = (r0, r1)
    lsems = (l0, l1)
    gsems = (g0, g1)
    ssems = (s0, s1)

    def fire_idx(g):
        p = g % 2
        gbase = base + g * GROUP
        return pltpu.async_copy(
            c_hbm.at[pl.ds(gbase, GROUP)], cbufs[p], lsems[p])

    idx_copies = {0: fire_idx(0)}
    store_copies = {}

    for g in range(N_GROUPS):
        p = g % 2
        gbase = base + g * GROUP
        idx_copies.pop(g).wait()
        if g >= 2:
            store_copies.pop(g - 2).wait()
        gathers = [
            pltpu.async_copy(
                t_sh.at[cbufs[p].at[pl.ds(b * DMA_B, DMA_B)]],
                rbufs[p].at[pl.ds(b * DMA_B, DMA_B)],
                gsems[p],
            )
            for b in range(N_DMA)
        ]
        if g + 1 < N_GROUPS:
            idx_copies[g + 1] = fire_idx(g + 1)
        for cp in gathers:
            cp.wait()
        store_copies[g] = pltpu.async_copy(
            rbufs[p], out_hbm.at[pl.ds(gbase, GROUP)], ssems[p])

    for g in (N_GROUPS - 2, N_GROUPS - 1):
        store_copies.pop(g).wait()


def kernel(edge_attr, emb0, emb1, emb2):
    ea = edge_attr.astype(jnp.int32)
    c = _fuse_index(ea.reshape(IDX_GRID, IDX_BE, 3)).reshape(E)
    t = _build_table(emb0, emb1, emb2)
    return _sc_lookup(c, t)


# speedup vs baseline: 2.7593x; 2.7593x over previous
import functools

import jax
import jax.numpy as jnp
from jax import lax
from jax.experimental import pallas as pl
from jax.experimental.pallas import tpu as pltpu
from jax.experimental.pallas import tpu_sc as plsc

E = 320000
D = 128
NCAT = 14
T_ROWS = NCAT * NCAT * NCAT
T_PAD = 2816

NC = 2
NS = 16
NW = NC * NS
R_PER_W = E // NW
GROUP = 400
N_GROUPS = R_PER_W // GROUP
DMA_B = 80
N_DMA = GROUP // DMA_B
JSTEPS = GROUP // 16


def _build_table(e0, e1, e2):
    def body(e0_ref, e1_ref, e2_ref, t_ref):
        t_ref[...] = (
            e0_ref[...][:, None, None, :]
            + e1_ref[...][None, :, None, :]
        ) + e2_ref[...][None, None, :, :]

    t4 = pl.pallas_call(
        body,
        out_shape=jax.ShapeDtypeStruct((NCAT, NCAT, NCAT, D), jnp.float32),
    )(e0, e1, e2)
    return t4.reshape(T_ROWS, D)


_mesh = plsc.VectorSubcoreMesh(core_axis_name="c", subcore_axis_name="s")


@functools.partial(
    pl.kernel,
    mesh=_mesh,
    out_type=jax.ShapeDtypeStruct((E, D), jnp.float32),
    scratch_types=[
        pltpu.VMEM((GROUP,), jnp.int32),
        pltpu.VMEM((GROUP,), jnp.int32),
        pltpu.VMEM((GROUP,), jnp.int32),
        pltpu.VMEM((GROUP,), jnp.int32),
        pltpu.VMEM((GROUP,), jnp.int32),
        pltpu.VMEM((GROUP,), jnp.int32),
        pltpu.VMEM((N_DMA, DMA_B), jnp.int32),
        pltpu.VMEM((N_DMA, DMA_B), jnp.int32),
        pltpu.VMEM((GROUP, D), jnp.float32),
        pltpu.VMEM((GROUP, D), jnp.float32),
        pltpu.SemaphoreType.DMA,
        pltpu.SemaphoreType.DMA,
        pltpu.SemaphoreType.DMA,
        pltpu.SemaphoreType.DMA,
        pltpu.SemaphoreType.DMA,
        pltpu.SemaphoreType.DMA,
        pltpu.VMEM_SHARED((T_PAD, D), jnp.float32),
    ],
)
def _sc_lookup(ea0_hbm, ea1_hbm, ea2_hbm, t_hbm, out_hbm,
               e00, e01, e02, e10, e11, e12, c0, c1, r0, r1,
               l0, l1, g0, g1, s0, s1, t_sh):
    sid = lax.axis_index("s")
    wid = sid * NC + lax.axis_index("c")
    base = wid * R_PER_W

    stripe = T_PAD // NS
    pltpu.sync_copy(t_hbm.at[pl.ds(sid * stripe, stripe)],
                    t_sh.at[pl.ds(sid * stripe, stripe)])
    plsc.subcore_barrier()
    ebufs = ((e00, e01, e02), (e10, e11, e12))
    cbufs = (c0, c1)
    rbufs = (r0, r1)
    lsems = (l0, l1)
    gsems = (g0, g1)
    ssems = (s0, s1)

    def fire_cols(g):
        p = g % 2
        gbase = base + g * GROUP
        return [
            pltpu.async_copy(eah.at[pl.ds(gbase, GROUP)], ebufs[p][k],
                             lsems[p])
            for k, eah in enumerate((ea0_hbm, ea1_hbm, ea2_hbm))
        ]

    col_copies = {0: fire_cols(0)}
    store_copies = {}

    for g in range(N_GROUPS):
        p = g % 2
        gbase = base + g * GROUP
        for cp in col_copies.pop(g):
            cp.wait()
        for j in range(JSTEPS):
            a0 = ebufs[p][0][pl.ds(j * 16, 16)]
            a1 = ebufs[p][1][pl.ds(j * 16, 16)]
            a2 = ebufs[p][2][pl.ds(j * 16, 16)]
            c = (a0 * NCAT + a1) * NCAT + a2
            cbufs[p][j // 5, pl.ds((j % 5) * 16, 16)] = c
        if g >= 2:
            store_copies.pop(g - 2).wait()
        gathers = [
            pltpu.async_copy(
                t_sh.at[cbufs[p].at[b]],
                rbufs[p].at[pl.ds(b * DMA_B, DMA_B)],
                gsems[p],
            )
            for b in range(N_DMA)
        ]
        if g + 1 < N_GROUPS:
            col_copies[g + 1] = fire_cols(g + 1)
        for cp in gathers:
            cp.wait()
        store_copies[g] = pltpu.async_copy(
            rbufs[p], out_hbm.at[pl.ds(gbase, GROUP)], ssems[p])

    for g in (N_GROUPS - 2, N_GROUPS - 1):
        store_copies.pop(g).wait()


def kernel(edge_attr, emb0, emb1, emb2):
    ea = edge_attr.astype(jnp.int32)
    ea0 = ea[:, 0]
    ea1 = ea[:, 1]
    ea2 = ea[:, 2]
    t = _build_table(emb0, emb1, emb2)
    t = jnp.concatenate([t, jnp.zeros((T_PAD - T_ROWS, D), jnp.float32)])
    return _sc_lookup(ea0, ea1, ea2, t)


# speedup vs baseline: 2.7886x; 1.0106x over previous
import functools

import jax
import jax.numpy as jnp
from jax import lax
from jax.experimental import pallas as pl
from jax.experimental.pallas import tpu as pltpu
from jax.experimental.pallas import tpu_sc as plsc

E = 320000
D = 128
NCAT = 14
T_ROWS = NCAT * NCAT * NCAT
T_PAD = 2816

NC = 2
NS = 16
NW = NC * NS
R_PER_W = E // NW
GROUP = 400
N_GROUPS = R_PER_W // GROUP
DMA_B = 80
N_DMA = GROUP // DMA_B
JSTEPS = GROUP // 16


def _build_table(e0, e1, e2):
    def body(e0_ref, e1_ref, e2_ref, t_ref):
        t_ref[...] = (
            e0_ref[...][:, None, None, :]
            + e1_ref[...][None, :, None, :]
        ) + e2_ref[...][None, None, :, :]

    t4 = pl.pallas_call(
        body,
        out_shape=jax.ShapeDtypeStruct((NCAT, NCAT, NCAT, D), jnp.float32),
    )(e0, e1, e2)
    return t4.reshape(T_ROWS, D)


_mesh = plsc.VectorSubcoreMesh(core_axis_name="c", subcore_axis_name="s")


@functools.partial(
    pl.kernel,
    mesh=_mesh,
    out_type=jax.ShapeDtypeStruct((E, D), jnp.float32),
    scratch_types=[
        pltpu.VMEM((GROUP,), jnp.int32),
        pltpu.VMEM((GROUP,), jnp.int32),
        pltpu.VMEM((GROUP,), jnp.int32),
        pltpu.VMEM((GROUP,), jnp.int32),
        pltpu.VMEM((GROUP,), jnp.int32),
        pltpu.VMEM((GROUP,), jnp.int32),
        pltpu.VMEM((GROUP,), jnp.int32),
        pltpu.VMEM((GROUP,), jnp.int32),
        pltpu.VMEM((GROUP, D), jnp.float32),
        pltpu.VMEM((GROUP, D), jnp.float32),
        pltpu.SemaphoreType.DMA,
        pltpu.SemaphoreType.DMA,
        pltpu.SemaphoreType.DMA,
        pltpu.SemaphoreType.DMA,
        pltpu.SemaphoreType.DMA,
        pltpu.SemaphoreType.DMA,
        pltpu.VMEM_SHARED((T_PAD, D), jnp.float32),
    ],
)
def _sc_lookup(ea0_hbm, ea1_hbm, ea2_hbm, t_hbm, out_hbm,
               e00, e01, e02, e10, e11, e12, c0, c1, r0, r1,
               l0, l1, g0, g1, s0, s1, t_sh):
    sid = lax.axis_index("s")
    wid = sid * NC + lax.axis_index("c")
    base = wid * R_PER_W
    ebufs = ((e00, e01, e02), (e10, e11, e12))
    cbufs = (c0, c1)
    rbufs = (r0, r1)
    lsems = (l0, l1)
    gsems = (g0, g1)
    ssems = (s0, s1)

    def fire_cols(g):
        p = g % 2
        gbase = base + g * GROUP
        return [
            pltpu.async_copy(eah.at[pl.ds(gbase, GROUP)], ebufs[p][k],
                             lsems[p])
            for k, eah in enumerate((ea0_hbm, ea1_hbm, ea2_hbm))
        ]

    col_copies = {0: fire_cols(0)}
    stripe = T_PAD // NS
    pltpu.sync_copy(t_hbm.at[pl.ds(sid * stripe, stripe)],
                    t_sh.at[pl.ds(sid * stripe, stripe)])
    plsc.subcore_barrier()
    store_copies = {}

    for g in range(N_GROUPS):
        p = g % 2
        gbase = base + g * GROUP
        for cp in col_copies.pop(g):
            cp.wait()
        for j in range(JSTEPS):
            a0 = ebufs[p][0][pl.ds(j * 16, 16)]
            a1 = ebufs[p][1][pl.ds(j * 16, 16)]
            a2 = ebufs[p][2][pl.ds(j * 16, 16)]
            c = (a0 * NCAT + a1) * NCAT + a2
            cbufs[p][pl.ds(j * 16, 16)] = c
        if g >= 2:
            store_copies.pop(g - 2).wait()
        gather = pltpu.async_copy(
            t_sh.at[cbufs[p]], rbufs[p], gsems[p])
        if g + 1 < N_GROUPS:
            col_copies[g + 1] = fire_cols(g + 1)
        gather.wait()
        store_copies[g] = pltpu.async_copy(
            rbufs[p], out_hbm.at[pl.ds(gbase, GROUP)], ssems[p])

    for g in (N_GROUPS - 2, N_GROUPS - 1):
        store_copies.pop(g).wait()


def kernel(edge_attr, emb0, emb1, emb2):
    ea = edge_attr.astype(jnp.int32)
    ea0 = ea[:, 0]
    ea1 = ea[:, 1]
    ea2 = ea[:, 2]
    t = _build_table(emb0, emb1, emb2)
    t = jnp.concatenate([t, jnp.zeros((T_PAD - T_ROWS, D), jnp.float32)])
    return _sc_lookup(ea0, ea1, ea2, t)


# speedup vs baseline: 2.7930x; 1.0016x over previous
import functools

import jax
import jax.numpy as jnp
from jax import lax
from jax.experimental import pallas as pl
from jax.experimental.pallas import tpu as pltpu
from jax.experimental.pallas import tpu_sc as plsc

E = 320000
D = 128
NCAT = 14
T_ROWS = NCAT * NCAT * NCAT

NC = 2
NS = 16
NW = NC * NS
R_PER_W = E // NW
GROUP = 400
N_GROUPS = R_PER_W // GROUP
DMA_B = 80
N_DMA = GROUP // DMA_B
JSTEPS = GROUP // 16


def _build_table(e0, e1, e2):
    def body(e0_ref, e1_ref, e2_ref, t_ref):
        t_ref[...] = (
            e0_ref[...][:, None, None, :]
            + e1_ref[...][None, :, None, :]
        ) + e2_ref[...][None, None, :, :]

    t4 = pl.pallas_call(
        body,
        out_shape=jax.ShapeDtypeStruct((NCAT, NCAT, NCAT, D), jnp.float32),
    )(e0, e1, e2)
    return t4.reshape(T_ROWS, D)


_mesh = plsc.VectorSubcoreMesh(core_axis_name="c", subcore_axis_name="s")


@functools.partial(
    pl.kernel,
    mesh=_mesh,
    out_type=jax.ShapeDtypeStruct((E, D), jnp.float32),
    scratch_types=[
        pltpu.VMEM((GROUP,), jnp.int32),
        pltpu.VMEM((GROUP,), jnp.int32),
        pltpu.VMEM((GROUP,), jnp.int32),
        pltpu.VMEM((GROUP,), jnp.int32),
        pltpu.VMEM((GROUP,), jnp.int32),
        pltpu.VMEM((GROUP,), jnp.int32),
        pltpu.VMEM((GROUP,), jnp.int32),
        pltpu.VMEM((GROUP,), jnp.int32),
        pltpu.VMEM((GROUP, D), jnp.float32),
        pltpu.VMEM((GROUP, D), jnp.float32),
        pltpu.SemaphoreType.DMA,
        pltpu.SemaphoreType.DMA,
        pltpu.SemaphoreType.DMA,
        pltpu.SemaphoreType.DMA,
        pltpu.SemaphoreType.DMA,
        pltpu.SemaphoreType.DMA,
        pltpu.VMEM_SHARED((T_ROWS, D), jnp.float32),
    ],
)
def _sc_lookup(ea0_hbm, ea1_hbm, ea2_hbm, t_hbm, out_hbm,
               e00, e01, e02, e10, e11, e12, c0, c1, r0, r1,
               l0, l1, g0, g1, s0, s1, t_sh):
    sid = lax.axis_index("s")
    wid = sid * NC + lax.axis_index("c")
    base = wid * R_PER_W
    ebufs = ((e00, e01, e02), (e10, e11, e12))
    cbufs = (c0, c1)
    rbufs = (r0, r1)
    lsems = (l0, l1)
    gsems = (g0, g1)
    ssems = (s0, s1)

    def fire_cols(g):
        p = g % 2
        gbase = base + g * GROUP
        return [
            pltpu.async_copy(eah.at[pl.ds(gbase, GROUP)], ebufs[p][k],
                             lsems[p])
            for k, eah in enumerate((ea0_hbm, ea1_hbm, ea2_hbm))
        ]

    col_copies = {0: fire_cols(0)}

    @pl.when(sid < NS - 1)
    def _():
        pltpu.sync_copy(t_hbm.at[pl.ds(sid * 176, 176)],
                        t_sh.at[pl.ds(sid * 176, 176)])

    @pl.when(sid == NS - 1)
    def _():
        pltpu.sync_copy(t_hbm.at[pl.ds(2640, 104)],
                        t_sh.at[pl.ds(2640, 104)])

    plsc.subcore_barrier()
    store_copies = {}

    for g in range(N_GROUPS):
        p = g % 2
        gbase = base + g * GROUP
        for cp in col_copies.pop(g):
            cp.wait()
        for j in range(JSTEPS):
            a0 = ebufs[p][0][pl.ds(j * 16, 16)]
            a1 = ebufs[p][1][pl.ds(j * 16, 16)]
            a2 = ebufs[p][2][pl.ds(j * 16, 16)]
            c = (a0 * NCAT + a1) * NCAT + a2
            cbufs[p][pl.ds(j * 16, 16)] = c
        if g >= 2:
            store_copies.pop(g - 2).wait()
        gather = pltpu.async_copy(
            t_sh.at[cbufs[p]], rbufs[p], gsems[p])
        if g + 1 < N_GROUPS:
            col_copies[g + 1] = fire_cols(g + 1)
        gather.wait()
        store_copies[g] = pltpu.async_copy(
            rbufs[p], out_hbm.at[pl.ds(gbase, GROUP)], ssems[p])

    for g in (N_GROUPS - 2, N_GROUPS - 1):
        store_copies.pop(g).wait()


def kernel(edge_attr, emb0, emb1, emb2):
    ea = edge_attr.astype(jnp.int32)
    t = _build_table(emb0, emb1, emb2)
    return _sc_lookup(ea[:, 0], ea[:, 1], ea[:, 2], t)


# speedup vs baseline: 2.7952x; 1.0008x over previous
import functools

import jax
import jax.numpy as jnp
from jax import lax
from jax.experimental import pallas as pl
from jax.experimental.pallas import tpu as pltpu
from jax.experimental.pallas import tpu_sc as plsc

E = 320000
D = 128
NCAT = 14
T_ROWS = NCAT * NCAT * NCAT

NC = 2
NS = 16
NW = NC * NS
R_PER_W = E // NW
GROUP = 400
N_GROUPS = R_PER_W // GROUP
DMA_B = 80
N_DMA = GROUP // DMA_B
JSTEPS = GROUP // 16


def _build_table(e0, e1, e2):
    def body(e0_ref, e1_ref, e2_ref, t_ref):
        t_ref[...] = (
            e0_ref[...][:, None, None, :]
            + e1_ref[...][None, :, None, :]
        ) + e2_ref[...][None, None, :, :]

    t4 = pl.pallas_call(
        body,
        out_shape=jax.ShapeDtypeStruct((NCAT, NCAT, NCAT, D), jnp.float32),
    )(e0, e1, e2)
    return t4.reshape(T_ROWS, D)


_mesh = plsc.VectorSubcoreMesh(core_axis_name="c", subcore_axis_name="s")


@functools.partial(
    pl.kernel,
    mesh=_mesh,
    out_type=jax.ShapeDtypeStruct((E, D), jnp.float32),
    scratch_types=[
        pltpu.VMEM((GROUP,), jnp.int32),
        pltpu.VMEM((GROUP,), jnp.int32),
        pltpu.VMEM((GROUP,), jnp.int32),
        pltpu.VMEM((GROUP,), jnp.int32),
        pltpu.VMEM((GROUP,), jnp.int32),
        pltpu.VMEM((GROUP,), jnp.int32),
        pltpu.VMEM((GROUP,), jnp.int32),
        pltpu.VMEM((GROUP,), jnp.int32),
        pltpu.VMEM((GROUP, D), jnp.float32),
        pltpu.VMEM((GROUP, D), jnp.float32),
        pltpu.SemaphoreType.DMA,
        pltpu.SemaphoreType.DMA,
        pltpu.SemaphoreType.DMA,
        pltpu.SemaphoreType.DMA,
        pltpu.SemaphoreType.DMA,
        pltpu.SemaphoreType.DMA,
        pltpu.VMEM_SHARED((T_ROWS, D), jnp.float32),
    ],
)
def _sc_lookup(ea0_hbm, ea1_hbm, ea2_hbm, t_hbm, out_hbm,
               e00, e01, e02, e10, e11, e12, c0, c1, r0, r1,
               l0, l1, g0, g1, s0, s1, t_sh):
    sid = lax.axis_index("s")
    wid = sid * NC + lax.axis_index("c")
    base = wid * R_PER_W
    ebufs = ((e00, e01, e02), (e10, e11, e12))
    cbufs = (c0, c1)
    rbufs = (r0, r1)
    lsems = (l0, l1)
    gsems = (g0, g1)
    ssems = (s0, s1)

    def fire_cols(g):
        p = g % 2
        gbase = base + g * GROUP
        return [
            pltpu.async_copy(eah.at[pl.ds(gbase, GROUP)], ebufs[p][k],
                             lsems[p])
            for k, eah in enumerate((ea0_hbm, ea1_hbm, ea2_hbm))
        ]

    col_copies = {0: fire_cols(0)}

    @pl.when(sid < NS - 1)
    def _():
        pltpu.sync_copy(t_hbm.at[pl.ds(sid * 176, 176)],
                        t_sh.at[pl.ds(sid * 176, 176)])

    @pl.when(sid == NS - 1)
    def _():
        pltpu.sync_copy(t_hbm.at[pl.ds(2640, 104)],
                        t_sh.at[pl.ds(2640, 104)])

    plsc.subcore_barrier()
    store_copies = {}

    for g in range(N_GROUPS):
        p = g % 2
        gbase = base + g * GROUP
        for cp in col_copies.pop(g):
            cp.wait()
        for j in range(JSTEPS):
            a0 = ebufs[p][0][pl.ds(j * 16, 16)]
            a1 = ebufs[p][1][pl.ds(j * 16, 16)]
            a2 = ebufs[p][2][pl.ds(j * 16, 16)]
            c = (a0 * NCAT + a1) * NCAT + a2
            cbufs[p][pl.ds(j * 16, 16)] = c
        if g >= 2:
            store_copies.pop(g - 2).wait()
        gather = pltpu.async_copy(
            t_sh.at[cbufs[p]], rbufs[p], gsems[p])
        if g + 1 < N_GROUPS:
            col_copies[g + 1] = fire_cols(g + 1)
        gather.wait()
        store_copies[g] = pltpu.async_copy(
            rbufs[p], out_hbm.at[pl.ds(gbase, GROUP)], ssems[p])

    for g in (N_GROUPS - 2, N_GROUPS - 1):
        store_copies.pop(g).wait()


def kernel(edge_attr, emb0, emb1, emb2):
    ea = edge_attr.astype(jnp.int32).T
    t = _build_table(emb0, emb1, emb2)
    return _sc_lookup(ea[0], ea[1], ea[2], t)


# speedup vs baseline: 2.8100x; 1.0053x over previous
import functools

import jax
import jax.numpy as jnp
from jax import lax
from jax.experimental import pallas as pl
from jax.experimental.pallas import tpu as pltpu
from jax.experimental.pallas import tpu_sc as plsc

E = 320000
D = 128
NCAT = 14
T_ROWS = NCAT * NCAT * NCAT

NC = 2
NS = 16
NW = NC * NS
R_PER_W = E // NW
GROUP = 400
N_GROUPS = R_PER_W // GROUP
DMA_B = 80
N_DMA = GROUP // DMA_B
JSTEPS = GROUP // 16


def _build_table(e0, e1, e2):
    def body(e0_ref, e1_ref, e2_ref, t_ref):
        t_ref[...] = (
            e0_ref[...][:, None, None, :]
            + e1_ref[...][None, :, None, :]
        ) + e2_ref[...][None, None, :, :]

    t4 = pl.pallas_call(
        body,
        out_shape=jax.ShapeDtypeStruct((NCAT, NCAT, NCAT, D), jnp.float32),
    )(e0, e1, e2)
    return t4.reshape(T_ROWS, D)


_mesh = plsc.VectorSubcoreMesh(core_axis_name="c", subcore_axis_name="s")


@functools.partial(
    pl.kernel,
    mesh=_mesh,
    out_type=jax.ShapeDtypeStruct((E, D), jnp.float32),
    scratch_types=[
        pltpu.VMEM((GROUP,), jnp.int32),
        pltpu.VMEM((GROUP,), jnp.int32),
        pltpu.VMEM((GROUP,), jnp.int32),
        pltpu.VMEM((GROUP,), jnp.int32),
        pltpu.VMEM((GROUP,), jnp.int32),
        pltpu.VMEM((GROUP,), jnp.int32),
        pltpu.VMEM((GROUP,), jnp.int32),
        pltpu.VMEM((GROUP,), jnp.int32),
        pltpu.VMEM((GROUP, D), jnp.float32),
        pltpu.VMEM((GROUP, D), jnp.float32),
        pltpu.SemaphoreType.DMA,
        pltpu.SemaphoreType.DMA,
        pltpu.SemaphoreType.DMA,
        pltpu.SemaphoreType.DMA,
        pltpu.SemaphoreType.DMA,
        pltpu.SemaphoreType.DMA,
        pltpu.VMEM_SHARED((T_ROWS, D), jnp.float32),
    ],
)
def _sc_lookup(ea0_hbm, ea1_hbm, ea2_hbm, t_hbm, out_hbm,
               e00, e01, e02, e10, e11, e12, c0, c1, r0, r1,
               l0, l1, g0, g1, s0, s1, t_sh):
    sid = lax.axis_index("s")
    wid = sid * NC + lax.axis_index("c")
    base = wid * R_PER_W
    ebufs = ((e00, e01, e02), (e10, e11, e12))
    cbufs = (c0, c1)
    rbufs = (r0, r1)
    lsems = (l0, l1)
    gsems = (g0, g1)
    ssems = (s0, s1)

    def fire_cols(g):
        p = g % 2
        gbase = base + g * GROUP
        return [
            pltpu.async_copy(eah.at[pl.ds(gbase, GROUP)], ebufs[p][k],
                             lsems[p])
            for k, eah in enumerate((ea0_hbm, ea1_hbm, ea2_hbm))
        ]

    col_copies = {0: fire_cols(0), 1: fire_cols(1)}

    @pl.when(sid < NS - 1)
    def _():
        pltpu.sync_copy(t_hbm.at[pl.ds(sid * 176, 176)],
                        t_sh.at[pl.ds(sid * 176, 176)])

    @pl.when(sid == NS - 1)
    def _():
        pltpu.sync_copy(t_hbm.at[pl.ds(2640, 104)],
                        t_sh.at[pl.ds(2640, 104)])

    plsc.subcore_barrier()
    store_copies = {}

    def compute_idx(g):
        p = g % 2
        for cp in col_copies.pop(g):
            cp.wait()
        for j in range(JSTEPS):
            a0 = ebufs[p][0][pl.ds(j * 16, 16)]
            a1 = ebufs[p][1][pl.ds(j * 16, 16)]
            a2 = ebufs[p][2][pl.ds(j * 16, 16)]
            c = (a0 * NCAT + a1) * NCAT + a2
            cbufs[p][pl.ds(j * 16, 16)] = c

    compute_idx(0)

    for g in range(N_GROUPS):
        p = g % 2
        gbase = base + g * GROUP
        if g >= 2:
            store_copies.pop(g - 2).wait()
        gather = pltpu.async_copy(
            t_sh.at[cbufs[p]], rbufs[p], gsems[p])
        if g + 2 < N_GROUPS:
            col_copies[g + 2] = fire_cols(g + 2)
        if g + 1 < N_GROUPS:
            compute_idx(g + 1)
        gather.wait()
        store_copies[g] = pltpu.async_copy(
            rbufs[p], out_hbm.at[pl.ds(gbase, GROUP)], ssems[p])

    for g in (N_GROUPS - 2, N_GROUPS - 1):
        store_copies.pop(g).wait()


def kernel(edge_attr, emb0, emb1, emb2):
    ea = edge_attr.astype(jnp.int32).T
    t = _build_table(emb0, emb1, emb2)
    return _sc_lookup(ea[0], ea[1], ea[2], t)


# speedup vs baseline: 2.8313x; 1.0076x over previous
import functools

import jax
import jax.numpy as jnp
from jax import lax
from jax.experimental import pallas as pl
from jax.experimental.pallas import tpu as pltpu
from jax.experimental.pallas import tpu_sc as plsc

E = 320000
D = 128
NCAT = 14
T_ROWS = NCAT * NCAT * NCAT

NC = 2
NS = 16
NW = NC * NS
R_PER_W = E // NW
GROUP = 400
N_GROUPS = R_PER_W // GROUP
DMA_B = 80
N_DMA = GROUP // DMA_B
JSTEPS = GROUP // 16


def _build_table(e0, e1, e2):
    def body(e0_ref, e1_ref, e2_ref, t_ref):
        i = pl.program_id(0)
        y1 = jnp.concatenate(
            [jnp.broadcast_to(e1_ref[k:k + 1, :], (NCAT, D))
             for k in range(NCAT)], 0)
        y2 = jnp.concatenate([e2_ref[...]] * NCAT, 0)
        t_ref[...] = jnp.concatenate(
            [(jnp.broadcast_to(e0_ref[pl.ds(2 * i + k, 1), :],
                               (NCAT * NCAT, D)) + y1)
             + y2 for k in range(2)], 0)

    return pl.pallas_call(
        body,
        grid=(NCAT // 2,),
        in_specs=[
            pl.BlockSpec((NCAT, D), lambda i: (0, 0)),
            pl.BlockSpec((NCAT, D), lambda i: (0, 0)),
            pl.BlockSpec((NCAT, D), lambda i: (0, 0)),
        ],
        out_specs=pl.BlockSpec((2 * NCAT * NCAT, D), lambda i: (i, 0)),
        out_shape=jax.ShapeDtypeStruct((T_ROWS, D), jnp.float32),
    )(e0, e1, e2)


_mesh = plsc.VectorSubcoreMesh(core_axis_name="c", subcore_axis_name="s")


@functools.partial(
    pl.kernel,
    mesh=_mesh,
    out_type=jax.ShapeDtypeStruct((E, D), jnp.float32),
    scratch_types=[
        pltpu.VMEM((GROUP,), jnp.int32),
        pltpu.VMEM((GROUP,), jnp.int32),
        pltpu.VMEM((GROUP,), jnp.int32),
        pltpu.VMEM((GROUP,), jnp.int32),
        pltpu.VMEM((GROUP,), jnp.int32),
        pltpu.VMEM((GROUP,), jnp.int32),
        pltpu.VMEM((GROUP,), jnp.int32),
        pltpu.VMEM((GROUP,), jnp.int32),
        pltpu.VMEM((GROUP, D), jnp.float32),
        pltpu.VMEM((GROUP, D), jnp.float32),
        pltpu.SemaphoreType.DMA,
        pltpu.SemaphoreType.DMA,
        pltpu.SemaphoreType.DMA,
        pltpu.SemaphoreType.DMA,
        pltpu.SemaphoreType.DMA,
        pltpu.SemaphoreType.DMA,
        pltpu.VMEM_SHARED((T_ROWS, D), jnp.float32),
    ],
)
def _sc_lookup(ea0_hbm, ea1_hbm, ea2_hbm, t_hbm, out_hbm,
               e00, e01, e02, e10, e11, e12, c0, c1, r0, r1,
               l0, l1, g0, g1, s0, s1, t_sh):
    sid = lax.axis_index("s")
    wid = sid * NC + lax.axis_index("c")
    base = wid * R_PER_W
    ebufs = ((e00, e01, e02), (e10, e11, e12))
    cbufs = (c0, c1)
    rbufs = (r0, r1)
    lsems = (l0, l1)
    gsems = (g0, g1)
    ssems = (s0, s1)

    def fire_cols(g):
        p = g % 2
        gbase = base + g * GROUP
        return [
            pltpu.async_copy(eah.at[pl.ds(gbase, GROUP)], ebufs[p][k],
                             lsems[p])
            for k, eah in enumerate((ea0_hbm, ea1_hbm, ea2_hbm))
        ]

    col_copies = {0: fire_cols(0), 1: fire_cols(1)}

    @pl.when(sid < NS - 1)
    def _():
        pltpu.sync_copy(t_hbm.at[pl.ds(sid * 176, 176)],
                        t_sh.at[pl.ds(sid * 176, 176)])

    @pl.when(sid == NS - 1)
    def _():
        pltpu.sync_copy(t_hbm.at[pl.ds(2640, 104)],
                        t_sh.at[pl.ds(2640, 104)])

    plsc.subcore_barrier()
    store_copies = {}

    def compute_idx(g):
        p = g % 2
        for cp in col_copies.pop(g):
            cp.wait()
        for j in range(JSTEPS):
            a0 = ebufs[p][0][pl.ds(j * 16, 16)]
            a1 = ebufs[p][1][pl.ds(j * 16, 16)]
            a2 = ebufs[p][2][pl.ds(j * 16, 16)]
            c = (a0 * NCAT + a1) * NCAT + a2
            cbufs[p][pl.ds(j * 16, 16)] = c

    compute_idx(0)

    for g in range(N_GROUPS):
        p = g % 2
        gbase = base + g * GROUP
        if g >= 2:
            store_copies.pop(g - 2).wait()
        gather = pltpu.async_copy(
            t_sh.at[cbufs[p]], rbufs[p], gsems[p])
        if g + 2 < N_GROUPS:
            col_copies[g + 2] = fire_cols(g + 2)
        if g + 1 < N_GROUPS:
            compute_idx(g + 1)
        gather.wait()
        store_copies[g] = pltpu.async_copy(
            rbufs[p], out_hbm.at[pl.ds(gbase, GROUP)], ssems[p])

    for g in (N_GROUPS - 2, N_GROUPS - 1):
        store_copies.pop(g).wait()


def kernel(edge_attr, emb0, emb1, emb2):
    ea = edge_attr.astype(jnp.int32).T
    t = _build_table(emb0, emb1, emb2)
    return _sc_lookup(ea[0], ea[1], ea[2], t)


# speedup vs baseline: 2.8403x; 1.0032x over previous
import functools

import jax
import jax.numpy as jnp
from jax import lax
from jax.experimental import pallas as pl
from jax.experimental.pallas import tpu as pltpu
from jax.experimental.pallas import tpu_sc as plsc

E = 320000
D = 128
NCAT = 14
T_ROWS = NCAT * NCAT * NCAT

NC = 2
NS = 16
NW = NC * NS
R_PER_W = E // NW
GROUP = 400
N_GROUPS = R_PER_W // GROUP
JSTEPS = GROUP // 16


def _build_table(e0, e1, e2):
    def body(e0_ref, e1_ref, e2_ref, t_ref):
        i = pl.program_id(0)
        y1 = jnp.concatenate(
            [jnp.broadcast_to(e1_ref[k:k + 1, :], (NCAT, D))
             for k in range(NCAT)], 0)
        y2 = jnp.concatenate([e2_ref[...]] * NCAT, 0)
        t_ref[...] = jnp.concatenate(
            [(jnp.broadcast_to(e0_ref[pl.ds(2 * i + k, 1), :],
                               (NCAT * NCAT, D)) + y1)
             + y2 for k in range(2)], 0)

    return pl.pallas_call(
        body,
        grid=(NCAT // 2,),
        in_specs=[
            pl.BlockSpec((NCAT, D), lambda i: (0, 0)),
            pl.BlockSpec((NCAT, D), lambda i: (0, 0)),
            pl.BlockSpec((NCAT, D), lambda i: (0, 0)),
        ],
        out_specs=pl.BlockSpec((2 * NCAT * NCAT, D), lambda i: (i, 0)),
        out_shape=jax.ShapeDtypeStruct((T_ROWS, D), jnp.float32),
    )(e0, e1, e2)


_mesh = plsc.VectorSubcoreMesh(core_axis_name="c", subcore_axis_name="s")


@functools.partial(
    pl.kernel,
    mesh=_mesh,
    out_type=jax.ShapeDtypeStruct((E, D), jnp.float32),
    scratch_types=[
        pltpu.VMEM((GROUP,), jnp.int32),
        pltpu.VMEM((GROUP,), jnp.int32),
        pltpu.VMEM((GROUP,), jnp.int32),
        pltpu.VMEM((GROUP,), jnp.int32),
        pltpu.VMEM((GROUP,), jnp.int32),
        pltpu.VMEM((GROUP,), jnp.int32),
        pltpu.VMEM((GROUP,), jnp.int32),
        pltpu.VMEM((GROUP,), jnp.int32),
        pltpu.VMEM((GROUP, D), jnp.float32),
        pltpu.VMEM((GROUP, D), jnp.float32),
        pltpu.SemaphoreType.DMA,
        pltpu.SemaphoreType.DMA,
        pltpu.SemaphoreType.DMA,
        pltpu.SemaphoreType.DMA,
        pltpu.SemaphoreType.DMA,
        pltpu.SemaphoreType.DMA,
        pltpu.VMEM_SHARED((T_ROWS, D), jnp.float32),
    ],
)
def _sc_lookup(ea0_hbm, ea1_hbm, ea2_hbm, t_hbm, out_hbm,
               e00, e01, e02, e10, e11, e12, c0, c1, r0, r1,
               l0, l1, g0, g1, s0, s1, t_sh):
    sid = lax.axis_index("s")
    wid = sid * NC + lax.axis_index("c")
    base = wid * R_PER_W
    ebufs = ((e00, e01, e02), (e10, e11, e12))
    cbufs = (c0, c1)
    rbufs = (r0, r1)
    lsems = (l0, l1)
    gsems = (g0, g1)
    ssems = (s0, s1)

    def fire_cols(g):
        p = g % 2
        gbase = base + g * GROUP
        return [
            pltpu.async_copy(eah.at[pl.ds(gbase, GROUP)], ebufs[p][k],
                             lsems[p])
            for k, eah in enumerate((ea0_hbm, ea1_hbm, ea2_hbm))
        ]

    col_copies = {0: fire_cols(0), 1: fire_cols(1)}

    @pl.when(sid < NS - 1)
    def _():
        pltpu.sync_copy(t_hbm.at[pl.ds(sid * 176, 176)],
                        t_sh.at[pl.ds(sid * 176, 176)])

    @pl.when(sid == NS - 1)
    def _():
        pltpu.sync_copy(t_hbm.at[pl.ds(2640, 104)],
                        t_sh.at[pl.ds(2640, 104)])

    plsc.subcore_barrier()
    store_copies = {}

    def compute_idx(g):
        p = g % 2
        for cp in col_copies.pop(g):
            cp.wait()
        for j in range(JSTEPS):
            a0 = ebufs[p][0][pl.ds(j * 16, 16)]
            a1 = ebufs[p][1][pl.ds(j * 16, 16)]
            a2 = ebufs[p][2][pl.ds(j * 16, 16)]
            c = (a0 * NCAT + a1) * NCAT + a2
            cbufs[p][pl.ds(j * 16, 16)] = c

    compute_idx(0)

    for g in range(N_GROUPS):
        p = g % 2
        gbase = base + g * GROUP
        if g >= 2:
            store_copies.pop(g - 2).wait()
        gather = pltpu.async_copy(
            t_sh.at[cbufs[p]], rbufs[p], gsems[p])
        if g + 2 < N_GROUPS:
            col_copies[g + 2] = fire_cols(g + 2)
        if g + 1 < N_GROUPS:
            compute_idx(g + 1)
        gather.wait()
        store_copies[g] = pltpu.async_copy(
            rbufs[p], out_hbm.at[pl.ds(gbase, GROUP)], ssems[p])

    for g in (N_GROUPS - 2, N_GROUPS - 1):
        store_copies.pop(g).wait()


def kernel(edge_attr, emb0, emb1, emb2):
    ea = edge_attr.astype(jnp.int32).T
    t = _build_table(emb0, emb1, emb2)
    return _sc_lookup(ea[0], ea[1], ea[2], t)
